# baseline (speedup 1.0000x reference)
import jax
import jax.numpy as jnp
from jax import lax
from jax.experimental import pallas as pl
from jax.experimental.pallas import tpu as pltpu
from jax.experimental.pallas import tpu_sc as plsc

_L = 16
_ROWS, _N = 128, 32768
_CHUNKS = _N // _L
_CAPL = 1024
_NEWTON_ITERS = 12
_NEG = -3.0e38


def _sc_body(x_hbm, o_hbm, row_v, cand_v):
    info = plsc.get_sparse_core_info()
    nc, ns = info.num_cores, info.num_subcores
    rpw = _ROWS // (nc * ns)
    wid = lax.axis_index("s") * nc + lax.axis_index("c")
    lane = lax.iota(jnp.int32, _L)
    base = lane * _CAPL

    for r in range(rpw):
        row = wid * rpw + r
        pltpu.sync_copy(x_hbm.at[row], row_v)

        def scan_body(i, carry):
            rm, off = carry
            v = row_v[pl.ds(i * _L, _L)]
            keep = v > rm - 1.0
            idx = base + jnp.minimum(off, _CAPL - 1)
            plsc.store_scatter(cand_v, [idx], v, mask=keep)
            rm = jnp.maximum(rm, v)
            off = off + jnp.where(keep, 1, 0)
            return rm, off

        rm0 = jnp.full((_L,), _NEG, jnp.float32)
        off0 = jnp.zeros((_L,), jnp.int32)
        rm, cnt = lax.fori_loop(0, _CHUNKS, scan_body, (rm0, off0))

        m = jnp.max(rm)
        nch = jnp.max(cnt)

        def fill_body(j, _):
            plsc.store_scatter(cand_v, [base + j], jnp.full((_L,), _NEG),
                               mask=j >= cnt)
            return 0
        lax.fori_loop(0, nch, fill_body, 0)

        def newton(_, t):
            def ch(j, acc):
                s, k = acc
                v = plsc.load_gather(cand_v, [base + j])
                act = v > t
                s = s + jnp.where(act, v, 0.0)
                k = k + jnp.where(act, 1.0, 0.0)
                return s, k
            z = jnp.zeros((_L,), jnp.float32)
            s, k = lax.fori_loop(0, nch, ch, (z, z))
            sv = jnp.broadcast_to(jnp.sum(s) - 1.0, (_L,))
            kv = jnp.broadcast_to(jnp.sum(k), (_L,))
            return sv / kv

        tau0 = jnp.broadcast_to(m - 1.0, (_L,))
        tau = lax.fori_loop(0, _NEWTON_ITERS, newton, tau0)

        def out_body(i, _):
            v = row_v[pl.ds(i * _L, _L)]
            row_v[pl.ds(i * _L, _L)] = jnp.maximum(v - tau, 0.0)
            return 0
        lax.fori_loop(0, _CHUNKS, out_body, 0)
        pltpu.sync_copy(row_v, o_hbm.at[row])


@jax.jit
def kernel(input_tensor):
    mesh = plsc.VectorSubcoreMesh(core_axis_name="c", subcore_axis_name="s")
    return pl.kernel(
        _sc_body,
        out_type=jax.ShapeDtypeStruct((_ROWS, _N), jnp.float32),
        mesh=mesh,
        scratch_types=[
            pltpu.VMEM((_N,), jnp.float32),
            pltpu.VMEM((_L * _CAPL,), jnp.float32),
        ],
        compiler_params=pltpu.CompilerParams(needs_layout_passes=False),
    )(input_tensor)


# speedup vs baseline: 1.9710x; 1.9710x over previous
import jax
import jax.numpy as jnp
from jax import lax
from jax.experimental import pallas as pl
from jax.experimental.pallas import tpu as pltpu
from jax.experimental.pallas import tpu_sc as plsc

_L = 16
_ROWS, _N = 128, 32768
_CHUNKS = _N // _L
_CAPL = 1024
_CAPL2 = 256
_UN = 8
_UN2 = 4
_NEWTON_ITERS = 12
_NEG = -3.0e38


def _sc_body(x_hbm, o_hbm, row_v, cand_v, cand2_v):
    info = plsc.get_sparse_core_info()
    nc, ns = info.num_cores, info.num_subcores
    rpw = _ROWS // (nc * ns)
    wid = lax.axis_index("s") * nc + lax.axis_index("c")
    lane = lax.iota(jnp.int32, _L)
    base = lane * _CAPL
    base2 = lane * _CAPL2
    ones = jnp.ones((_L,), jnp.int32)
    sent = jnp.full((_L,), _NEG, jnp.float32)

    for r in range(rpw):
        row = wid * rpw + r
        pltpu.sync_copy(x_hbm.at[row], row_v)

        def scan_body(i, carry):
            rm, off = carry
            for u in range(_UN):
                v = row_v[pl.ds((i * _UN + u) * _L, _L)]
                keep = v > rm - 1.0
                idx = base + jnp.minimum(off, _CAPL - 1)
                plsc.store_scatter(cand_v, [idx], v, mask=keep)
                rm = jnp.maximum(rm, v)
                off = off + jnp.where(keep, ones, 0)
            return rm, off

        rm0 = jnp.full((_L,), _NEG, jnp.float32)
        off0 = jnp.zeros((_L,), jnp.int32)
        rm, cnt = lax.fori_loop(0, _CHUNKS // _UN, scan_body, (rm0, off0))

        m = jnp.max(rm)
        mv = jnp.broadcast_to(m, (_L,))
        nch = jnp.max(cnt)

        def rec_body(i, off2):
            for u in range(_UN2):
                j = i * _UN2 + u
                v = plsc.load_gather(cand_v, [base + j])
                valid = (j < cnt) & (v > mv - 1.0)
                idx2 = base2 + jnp.minimum(off2, _CAPL2 - 1)
                plsc.store_scatter(cand2_v, [idx2], v, mask=valid)
                off2 = off2 + jnp.where(valid, ones, 0)
            return off2

        n_rec = (nch + (_UN2 - 1)) // _UN2
        cnt2 = lax.fori_loop(0, n_rec, rec_body, off0)

        nch2 = jnp.max(cnt2)
        n_new = (nch2 + (_UN2 - 1)) // _UN2

        def fill_body(j, _):
            plsc.store_scatter(cand2_v, [base2 + jnp.minimum(j, _CAPL2 - 1)],
                               sent, mask=j >= cnt2)
            return 0
        lax.fori_loop(0, n_new * _UN2, fill_body, 0)

        def newton(_, t):
            def ch(i, acc):
                s, k = acc
                for u in range(_UN2):
                    v = plsc.load_gather(cand2_v, [base2 + (i * _UN2 + u)])
                    act = v > t
                    s = s + jnp.where(act, v, 0.0)
                    k = k + jnp.where(act, 1.0, 0.0)
                return s, k
            z = jnp.zeros((_L,), jnp.float32)
            s, k = lax.fori_loop(0, n_new, ch, (z, z))
            sv = jnp.broadcast_to(jnp.sum(s) - 1.0, (_L,))
            kv = jnp.broadcast_to(jnp.sum(k), (_L,))
            return sv / kv

        tau0 = jnp.broadcast_to(m - 1.0, (_L,))
        tau = lax.fori_loop(0, _NEWTON_ITERS, newton, tau0)

        def out_body(i, _):
            for u in range(_UN):
                sl = pl.ds((i * _UN + u) * _L, _L)
                row_v[sl] = jnp.maximum(row_v[sl] - tau, 0.0)
            return 0
        lax.fori_loop(0, _CHUNKS // _UN, out_body, 0)
        pltpu.sync_copy(row_v, o_hbm.at[row])


@jax.jit
def kernel(input_tensor):
    mesh = plsc.VectorSubcoreMesh(core_axis_name="c", subcore_axis_name="s")
    return pl.kernel(
        _sc_body,
        out_type=jax.ShapeDtypeStruct((_ROWS, _N), jnp.float32),
        mesh=mesh,
        scratch_types=[
            pltpu.VMEM((_N,), jnp.float32),
            pltpu.VMEM((_L * _CAPL,), jnp.float32),
            pltpu.VMEM((_L * _CAPL2,), jnp.float32),
        ],
        compiler_params=pltpu.CompilerParams(needs_layout_passes=False),
    )(input_tensor)
